# baseline (speedup 1.0000x reference)
import jax
import jax.numpy as jnp
from jax import lax
from jax.experimental import pallas as pl
from jax.experimental.pallas import tpu as pltpu
from jax.experimental.pallas import tpu_sc as plsc

_EMB = 64
_SCALE = 8.0

_N = 4096 * 200
_NW = 32
_PW = _N // _NW
_C = 512
_G = 128
_KG = _C // _G
_NCH = _PW // _C


def _sc_embed(tok_hbm, table_hbm, out_hbm, idx_v, rows_v, gsem, osem):
    wid = lax.axis_index("s") * 2 + lax.axis_index("c")
    trow0 = wid * (_PW // _G)
    obase = wid * _PW

    def start_gather(c, b):
        pltpu.sync_copy(tok_hbm.at[pl.ds(trow0 + c * _KG, _KG)], idx_v.at[b])
        for j in range(_KG):
            pltpu.async_copy(
                table_hbm.at[idx_v.at[b, j]],
                rows_v.at[b, pl.ds(j * _G, _G)],
                gsem.at[b],
            )

    def drain_gather(b):
        pltpu.make_async_copy(
            table_hbm.at[pl.ds(0, _C)], rows_v.at[b], gsem.at[b]
        ).wait()

    def drain_out(b):
        pltpu.make_async_copy(
            rows_v.at[b], out_hbm.at[pl.ds(0, _C)], osem.at[b]
        ).wait()

    def scale(b):
        def body(i, carry):
            for k in range(_EMB // 16):
                sl = pl.ds(k * 16, 16)
                rows_v[b, i, sl] = rows_v[b, i, sl] * _SCALE
            return carry
        lax.fori_loop(0, _C, body, 0)

    def start_out(c, b):
        pltpu.async_copy(
            rows_v.at[b], out_hbm.at[pl.ds(obase + c * _C, _C)], osem.at[b]
        )

    start_gather(0, 0)

    def step(k, carry):
        for b in range(2):
            c = k * 2 + b
            b2 = 1 - b
            drain_gather(b)

            @pl.when(c + 1 < _NCH)
            def _prefetch():
                @pl.when(c >= 1)
                def _free_buf():
                    drain_out(b2)
                start_gather(c + 1, b2)

            scale(b)
            start_out(c, b)
        return carry

    lax.fori_loop(0, _NCH // 2, step, 0)
    drain_out(0)
    drain_out(1)


def kernel(tokens, table):
    tok2d = tokens.astype(jnp.int32).reshape(_N // _G, _G)
    mesh = plsc.VectorSubcoreMesh(core_axis_name="c", subcore_axis_name="s")
    out = pl.kernel(
        _sc_embed,
        out_type=jax.ShapeDtypeStruct((_N, _EMB), jnp.float32),
        mesh=mesh,
        scratch_types=[
            pltpu.VMEM((2, _KG, _G), jnp.int32),
            pltpu.VMEM((2, _C, _EMB), jnp.float32),
            pltpu.SemaphoreType.DMA((2,)),
            pltpu.SemaphoreType.DMA((2,)),
        ],
        compiler_params=pltpu.CompilerParams(use_tc_tiling_on_sc=False),
    )(tok2d, table)
    return out.reshape(tokens.shape[0], tokens.shape[1], _EMB)


# speedup vs baseline: 1.0580x; 1.0580x over previous
import jax
import jax.numpy as jnp
from jax import lax
from jax.experimental import pallas as pl
from jax.experimental.pallas import tpu as pltpu
from jax.experimental.pallas import tpu_sc as plsc

_EMB = 64
_SCALE = 8.0

_N = 4096 * 200
_NW = 32
_PW = _N // _NW
_C = 640
_G = 128
_KG = _C // _G
_NCH = _PW // _C


def _sc_embed(tok_hbm, table_hbm, out_hbm, idx_v, rows_v, gsem, osem):
    wid = lax.axis_index("s") * 2 + lax.axis_index("c")
    trow0 = wid * (_PW // _G)
    obase = wid * _PW

    pltpu.sync_copy(tok_hbm.at[pl.ds(trow0, _PW // _G)], idx_v)

    def start_gather(c, b):
        for j in range(_KG):
            pltpu.async_copy(
                table_hbm.at[idx_v.at[c * _KG + j]],
                rows_v.at[b, pl.ds(j * _G, _G)],
                gsem.at[b],
            )

    def drain_gather(b):
        pltpu.make_async_copy(
            table_hbm.at[pl.ds(0, _C)], rows_v.at[b], gsem.at[b]
        ).wait()

    def drain_out(b):
        pltpu.make_async_copy(
            rows_v.at[b], out_hbm.at[pl.ds(0, _C)], osem.at[b]
        ).wait()

    def scale(b):
        def body(i, carry):
            for r in range(2):
                for k in range(_EMB // 16):
                    sl = pl.ds(k * 16, 16)
                    rows_v[b, i * 2 + r, sl] = rows_v[b, i * 2 + r, sl] * _SCALE
            return carry
        lax.fori_loop(0, _C // 2, body, 0)

    def start_out(c, b):
        pltpu.async_copy(
            rows_v.at[b], out_hbm.at[pl.ds(obase + c * _C, _C)], osem.at[b]
        )

    start_gather(0, 0)

    def step(k, carry):
        for b in range(2):
            c = k * 2 + b
            b2 = 1 - b
            drain_gather(b)

            @pl.when(c + 1 < _NCH)
            def _prefetch():
                @pl.when(c >= 1)
                def _free_buf():
                    drain_out(b2)
                start_gather(c + 1, b2)

            scale(b)
            start_out(c, b)
        return carry

    lax.fori_loop(0, _NCH // 2, step, 0)
    drain_out(0)
    drain_out(1)


def kernel(tokens, table):
    tok2d = tokens.astype(jnp.int32).reshape(_N // _G, _G)
    mesh = plsc.VectorSubcoreMesh(core_axis_name="c", subcore_axis_name="s")
    out = pl.kernel(
        _sc_embed,
        out_type=jax.ShapeDtypeStruct((_N, _EMB), jnp.float32),
        mesh=mesh,
        scratch_types=[
            pltpu.VMEM((_PW // _G, _G), jnp.int32),
            pltpu.VMEM((2, _C, _EMB), jnp.float32),
            pltpu.SemaphoreType.DMA((2,)),
            pltpu.SemaphoreType.DMA((2,)),
        ],
        compiler_params=pltpu.CompilerParams(use_tc_tiling_on_sc=False),
    )(tok2d, table)
    return out.reshape(tokens.shape[0], tokens.shape[1], _EMB)
